# baseline (speedup 1.0000x reference)
import functools

import jax
import jax.numpy as jnp
from jax import lax
from jax.experimental import pallas as pl
from jax.experimental.pallas import tpu as pltpu
from jax.experimental.pallas import tpu_sc as plsc

_K = 1024
_L = 16
_NC = 2
_NS = 16
_NCHUNK = 14
_GROUP = 3
_BANK = _L * _K


def _sc_segment_max(imgf, spxf, *, n_rows, hw, n_batch):
    rows_per_w = n_rows // (_NC * _NS)
    ch = hw // _NCHUNK
    mesh = plsc.VectorSubcoreMesh(core_axis_name="c", subcore_axis_name="s")

    @functools.partial(
        pl.kernel,
        out_type=jax.ShapeDtypeStruct((n_rows, _K), jnp.float32),
        mesh=mesh,
        scratch_types=[
            pltpu.VMEM((ch,), jnp.int32),
            pltpu.VMEM((ch,), jnp.int32),
            pltpu.VMEM((ch,), jnp.float32),
            pltpu.VMEM((ch,), jnp.float32),
            pltpu.VMEM((ch,), jnp.float32),
            pltpu.VMEM((ch,), jnp.float32),
            pltpu.VMEM((ch,), jnp.float32),
            pltpu.VMEM((ch,), jnp.float32),
            pltpu.VMEM((_GROUP * _BANK,), jnp.float32),
            pltpu.VMEM((_K,), jnp.float32),
            pltpu.VMEM((_K,), jnp.float32),
            pltpu.VMEM((_K,), jnp.float32),
            pltpu.SemaphoreType.DMA,
            pltpu.SemaphoreType.DMA,
            pltpu.SemaphoreType.DMA,
            pltpu.SemaphoreType.DMA,
            pltpu.SemaphoreType.DMA,
            pltpu.SemaphoreType.DMA,
            pltpu.SemaphoreType.DMA,
            pltpu.SemaphoreType.DMA,
        ],
        compiler_params=pltpu.CompilerParams(needs_layout_passes=False),
    )
    def body(img_hbm, spx_hbm, out_hbm, idx0, idx1,
             bufa0, bufa1, bufb0, bufb1, bufc0, bufc1,
             acc_v, outa_v, outb_v, outc_v,
             isem0, isem1, sa0, sa1, sb0, sb1, sc0, sc1):
        cid = lax.axis_index("c")
        sid = lax.axis_index("s")
        wid = sid * _NC + cid
        row0 = wid * rows_per_w
        b = row0 // (n_rows // n_batch)

        lane = lax.iota(jnp.int32, _L)
        lane16 = lane * _L

        ninf = jnp.full((_L,), -jnp.inf, dtype=jnp.float32)

        @pl.loop(0, _GROUP * _K)
        def _init(j):
            acc_v[pl.ds(j * _L, _L)] = ninf

        idxs = (idx0, idx1)
        isems = (isem0, isem1)
        bufs = ((bufa0, bufa1), (bufb0, bufb1), (bufc0, bufc1))
        sems = ((sa0, sa1), (sb0, sb1), (sc0, sc1))
        outs = (outa_v, outb_v, outc_v)

        def issue(r0, c, par):
            pltpu.async_copy(
                spx_hbm.at[b, pl.ds(c * ch, ch)], idxs[par], isems[par])
            for k in range(_GROUP):
                pltpu.async_copy(
                    img_hbm.at[r0 + k, pl.ds(c * ch, ch)],
                    bufs[k][par], sems[k][par])

        def wait(r0, c, par):
            pltpu.make_async_copy(
                spx_hbm.at[b, pl.ds(c * ch, ch)],
                idxs[par], isems[par]).wait()
            for k in range(_GROUP):
                pltpu.make_async_copy(
                    img_hbm.at[r0 + k, pl.ds(c * ch, ch)],
                    bufs[k][par], sems[k][par]).wait()

        issue(row0, 0, 0)

        @pl.loop(0, rows_per_w // _GROUP)
        def _grp(g):
            r0 = row0 + _GROUP * g
            nr0 = jnp.minimum(r0 + _GROUP, n_rows - _GROUP)
            for c in range(_NCHUNK):
                cur, oth = c % 2, (c + 1) % 2
                if c + 1 < _NCHUNK:
                    issue(r0, c + 1, oth)
                else:
                    issue(nr0, 0, oth)
                wait(r0, c, cur)

                ib = idxs[cur]
                ba, bb, bc = bufs[0][cur], bufs[1][cur], bufs[2][cur]

                @pl.loop(0, ch // (4 * _L), unroll=2)
                def _scat(j):
                    s = j * (4 * _L)
                    i0 = ib[pl.ds(s, _L)] * _L + lane
                    i1 = ib[pl.ds(s + _L, _L)] * _L + lane
                    i2 = ib[pl.ds(s + 2 * _L, _L)] * _L + lane
                    i3 = ib[pl.ds(s + 3 * _L, _L)] * _L + lane
                    m10 = i1 == i0
                    m20 = i2 == i0
                    m21 = i2 == i1
                    m30 = i3 == i0
                    m31 = i3 == i1
                    m32 = i3 == i2
                    addrs = []
                    vals = []
                    gaths = []
                    for k, buf in enumerate((ba, bb, bc)):
                        o = k * _BANK
                        p = (i0 + o, i1 + o, i2 + o, i3 + o)
                        v = tuple(buf[pl.ds(s + t * _L, _L)]
                                  for t in range(4))
                        addrs.append(p)
                        vals.append(v)
                        gaths.append(tuple(
                            plsc.load_gather(acc_v, [p[t]])
                            for t in range(4)))
                    for k in range(_GROUP):
                        p = addrs[k]
                        v = vals[k]
                        g = gaths[k]
                        v1 = jnp.where(m10, jnp.maximum(v[1], v[0]), v[1])
                        v2 = jnp.where(m20, jnp.maximum(v[2], v[0]), v[2])
                        v2 = jnp.where(m21, jnp.maximum(v2, v[1]), v2)
                        v3 = jnp.where(m30, jnp.maximum(v[3], v[0]), v[3])
                        v3 = jnp.where(m31, jnp.maximum(v3, v[1]), v3)
                        v3 = jnp.where(m32, jnp.maximum(v3, v[2]), v3)
                        plsc.store_scatter(acc_v, [p[0]],
                                           jnp.maximum(g[0], v[0]))
                        plsc.store_scatter(acc_v, [p[1]],
                                           jnp.maximum(g[1], v1))
                        plsc.store_scatter(acc_v, [p[2]],
                                           jnp.maximum(g[2], v2))
                        plsc.store_scatter(acc_v, [p[3]],
                                           jnp.maximum(g[3], v3))

            @pl.loop(0, _K // _L)
            def _red(q):
                base = lane16 + q * (_L * _L)
                for k in range(_GROUP):
                    boff = k * _BANK
                    rot = lane
                    m = ninf
                    for t in range(_L):
                        g = plsc.load_gather(acc_v, [base + rot + boff])
                        m = jnp.maximum(m, g)
                        if t + 1 < _L:
                            rot = (rot + 1) & 15
                    outs[k][pl.ds(q * _L, _L)] = m

            @pl.loop(0, _GROUP * _K, unroll=4)
            def _reinit(j):
                acc_v[pl.ds(j * _L, _L)] = ninf

            for k in range(_GROUP):
                pltpu.sync_copy(outs[k], out_hbm.at[r0 + k])

        wait(row0, 0, 0)

    return body


def kernel(img, spx):
    B, C, H, W = img.shape
    hw = H * W
    imgf = img.reshape(B * C, hw)
    spxf = spx.reshape(B, hw).astype(jnp.int32)
    out = _sc_segment_max(imgf, spxf, n_rows=B * C, hw=hw, n_batch=B)(
        imgf, spxf)
    return out.reshape(B, C, _K)
